# baseline (speedup 1.0000x reference)
import functools

import jax
import jax.numpy as jnp
from jax import lax
from jax.experimental import pallas as pl
from jax.experimental.pallas import tpu as pltpu
from jax.experimental.pallas import tpu_sc as plsc

D_MODEL = 128
BATCH = 16384
_NUM_CORES = 2
_NUM_SUBCORES = 16
_NW = _NUM_CORES * _NUM_SUBCORES
_BPW = BATCH // _NW

_CH = 128
_NCH = _BPW // _CH

_mesh = plsc.VectorSubcoreMesh(core_axis_name="c", subcore_axis_name="s")


@functools.partial(
    pl.kernel,
    mesh=_mesh,
    out_type=jax.ShapeDtypeStruct((BATCH, D_MODEL), jnp.float32),
    scratch_types=[
        pltpu.VMEM((_BPW,), jnp.int32),
        pltpu.VMEM((2, _CH, D_MODEL), jnp.float32),
        pltpu.SemaphoreType.DMA,
        pltpu.SemaphoreType.DMA,
        pltpu.SemaphoreType.DMA,
        pltpu.SemaphoreType.DMA,
    ],
)
def _pe_gather(t_hbm, pe_hbm, out_hbm, idx_v, rows_v, g0, g1, s0, s1):
    gsems = (g0, g1)
    ssems = (s0, s1)
    wid = lax.axis_index("s") * _NUM_CORES + lax.axis_index("c")
    base = wid * _BPW
    pltpu.sync_copy(t_hbm.at[pl.ds(base, _BPW)], idx_v)
    gathers = [None] * _NCH
    scatters = [None] * _NCH
    gathers[0] = pltpu.async_copy(
        pe_hbm.at[idx_v.at[pl.ds(0, _CH)]], rows_v.at[0], gsems[0]
    )
    for c in range(_NCH):
        b = c % 2
        gathers[c].wait()
        scatters[c] = pltpu.async_copy(
            rows_v.at[b], out_hbm.at[pl.ds(base + c * _CH, _CH)], ssems[b]
        )
        if c + 1 < _NCH:
            if c >= 1:
                scatters[c - 1].wait()
            gathers[c + 1] = pltpu.async_copy(
                pe_hbm.at[idx_v.at[pl.ds((c + 1) * _CH, _CH)]],
                rows_v.at[(c + 1) % 2],
                gsems[(c + 1) % 2],
            )
    if _NCH >= 2:
        scatters[_NCH - 2].wait()
    scatters[_NCH - 1].wait()


def kernel(t, pe):
    return _pe_gather(t, pe)


# speedup vs baseline: 1.0050x; 1.0050x over previous
import functools

import jax
import jax.numpy as jnp
from jax import lax
from jax.experimental import pallas as pl
from jax.experimental.pallas import tpu as pltpu
from jax.experimental.pallas import tpu_sc as plsc

D_MODEL = 128
BATCH = 16384
_NUM_CORES = 2
_NUM_SUBCORES = 16
_NW = _NUM_CORES * _NUM_SUBCORES
_BPW = BATCH // _NW

_CH = _BPW // 2

_mesh = plsc.VectorSubcoreMesh(core_axis_name="c", subcore_axis_name="s")


@functools.partial(
    pl.kernel,
    mesh=_mesh,
    out_type=jax.ShapeDtypeStruct((BATCH, D_MODEL), jnp.float32),
    scratch_types=[
        pltpu.VMEM((_BPW,), jnp.int32),
        pltpu.VMEM((2, _CH, D_MODEL), jnp.float32),
        pltpu.SemaphoreType.DMA,
        pltpu.SemaphoreType.DMA,
        pltpu.SemaphoreType.DMA,
        pltpu.SemaphoreType.DMA,
    ],
)
def _pe_gather(t_hbm, pe_hbm, out_hbm, idx_v, rows_v, g0, g1, s0, s1):
    wid = lax.axis_index("s") * _NUM_CORES + lax.axis_index("c")
    base = wid * _BPW
    pltpu.sync_copy(t_hbm.at[pl.ds(base, _BPW)], idx_v)
    ga = pltpu.async_copy(pe_hbm.at[idx_v.at[pl.ds(0, _CH)]], rows_v.at[0], g0)
    gb = pltpu.async_copy(pe_hbm.at[idx_v.at[pl.ds(_CH, _CH)]], rows_v.at[1], g1)
    ga.wait()
    sa = pltpu.async_copy(rows_v.at[0], out_hbm.at[pl.ds(base, _CH)], s0)
    gb.wait()
    sb = pltpu.async_copy(rows_v.at[1], out_hbm.at[pl.ds(base + _CH, _CH)], s1)
    sa.wait()
    sb.wait()


def kernel(t, pe):
    return _pe_gather(t, pe)


# speedup vs baseline: 1.1349x; 1.1292x over previous
import functools

import jax
import jax.numpy as jnp
from jax import lax
from jax.experimental import pallas as pl
from jax.experimental.pallas import tpu as pltpu
from jax.experimental.pallas import tpu_sc as plsc

D_MODEL = 128
BATCH = 16384
_NUM_CORES = 2
_NUM_SUBCORES = 16
_NW = _NUM_CORES * _NUM_SUBCORES
_BPW = BATCH // _NW

_TABLE_ROWS = 1000

_mesh = plsc.VectorSubcoreMesh(core_axis_name="c", subcore_axis_name="s")


@functools.partial(
    pl.kernel,
    mesh=_mesh,
    out_type=jax.ShapeDtypeStruct((BATCH, D_MODEL), jnp.float32),
    scratch_types=[
        pltpu.VMEM((_BPW,), jnp.int32),
        pltpu.VMEM((_BPW, D_MODEL), jnp.float32),
        pltpu.VMEM_SHARED((_TABLE_ROWS, D_MODEL), jnp.float32),
        pltpu.SemaphoreType.DMA,
    ],
)
def _pe_gather(t_hbm, pe_hbm, out_hbm, idx_v, rows_v, table_s, sem):
    wid = lax.axis_index("s") * _NUM_CORES + lax.axis_index("c")
    base = wid * _BPW
    @pl.when(lax.axis_index("s") == 0)
    def _():
        pltpu.sync_copy(pe_hbm, table_s)

    pltpu.sync_copy(t_hbm.at[pl.ds(base, _BPW)], idx_v)
    plsc.subcore_barrier()
    pltpu.async_copy(table_s.at[idx_v], rows_v, sem).wait()
    pltpu.sync_copy(rows_v, out_hbm.at[pl.ds(base, _BPW)])


def kernel(t, pe):
    return _pe_gather(t, pe)


# speedup vs baseline: 1.1608x; 1.0228x over previous
import functools

import jax
import jax.numpy as jnp
from jax import lax
from jax.experimental import pallas as pl
from jax.experimental.pallas import tpu as pltpu
from jax.experimental.pallas import tpu_sc as plsc

D_MODEL = 128
BATCH = 16384
_NUM_CORES = 2
_NUM_SUBCORES = 16
_NW = _NUM_CORES * _NUM_SUBCORES
_BPW = BATCH // _NW

_TABLE_ROWS = 1000

_mesh = plsc.VectorSubcoreMesh(core_axis_name="c", subcore_axis_name="s")


@functools.partial(
    pl.kernel,
    mesh=_mesh,
    out_type=jax.ShapeDtypeStruct((BATCH, D_MODEL), jnp.float32),
    scratch_types=[
        pltpu.VMEM((_BPW,), jnp.int32),
        pltpu.VMEM((_BPW, D_MODEL), jnp.float32),
        pltpu.VMEM_SHARED((_TABLE_ROWS, D_MODEL), jnp.float32),
        pltpu.SemaphoreType.DMA,
    ],
)
def _pe_gather(t_hbm, pe_hbm, out_hbm, idx_v, rows_v, table_s, sem):
    sid = lax.axis_index("s")
    wid = sid * _NUM_CORES + lax.axis_index("c")
    base = wid * _BPW
    half = _BPW // 2
    @pl.when(sid < 7)
    def _():
        pltpu.sync_copy(
            pe_hbm.at[pl.ds(sid * 128, 128)], table_s.at[pl.ds(sid * 128, 128)]
        )

    @pl.when(sid == 7)
    def _():
        pltpu.sync_copy(pe_hbm.at[pl.ds(896, 104)], table_s.at[pl.ds(896, 104)])

    pltpu.sync_copy(t_hbm.at[pl.ds(base, _BPW)], idx_v)
    plsc.subcore_barrier()
    ga = pltpu.async_copy(
        table_s.at[idx_v.at[pl.ds(0, half)]], rows_v.at[pl.ds(0, half)], sem
    )
    gb = pltpu.async_copy(
        table_s.at[idx_v.at[pl.ds(half, half)]], rows_v.at[pl.ds(half, half)], sem
    )
    ga.wait()
    pltpu.sync_copy(rows_v.at[pl.ds(0, half)], out_hbm.at[pl.ds(base, half)])
    gb.wait()
    pltpu.sync_copy(rows_v.at[pl.ds(half, half)], out_hbm.at[pl.ds(base + half, half)])


def kernel(t, pe):
    return _pe_gather(t, pe)


# speedup vs baseline: 1.1756x; 1.0127x over previous
import functools

import jax
import jax.numpy as jnp
from jax import lax
from jax.experimental import pallas as pl
from jax.experimental.pallas import tpu as pltpu
from jax.experimental.pallas import tpu_sc as plsc

D_MODEL = 128
BATCH = 16384
_NUM_CORES = 2
_NUM_SUBCORES = 16
_NW = _NUM_CORES * _NUM_SUBCORES
_BPW = BATCH // _NW

_TABLE_ROWS = 1000
_K = 4

_mesh = plsc.VectorSubcoreMesh(core_axis_name="c", subcore_axis_name="s")


@functools.partial(
    pl.kernel,
    mesh=_mesh,
    out_type=jax.ShapeDtypeStruct((BATCH, D_MODEL), jnp.float32),
    scratch_types=[
        pltpu.VMEM((_BPW,), jnp.int32),
        pltpu.VMEM((_BPW, D_MODEL), jnp.float32),
        pltpu.VMEM_SHARED((_TABLE_ROWS, D_MODEL), jnp.float32),
        pltpu.SemaphoreType.DMA,
        pltpu.SemaphoreType.DMA,
    ],
)
def _pe_gather(t_hbm, pe_hbm, out_hbm, idx_v, rows_v, table_s, sem, ssem):
    sid = lax.axis_index("s")
    wid = sid * _NUM_CORES + lax.axis_index("c")
    base = wid * _BPW
    @pl.when(sid < 7)
    def _():
        pltpu.sync_copy(
            pe_hbm.at[pl.ds(sid * 128, 128)], table_s.at[pl.ds(sid * 128, 128)]
        )

    @pl.when(sid == 7)
    def _():
        pltpu.sync_copy(pe_hbm.at[pl.ds(896, 104)], table_s.at[pl.ds(896, 104)])

    pltpu.sync_copy(t_hbm.at[pl.ds(base, _BPW)], idx_v)
    plsc.subcore_barrier()
    ch = _BPW // _K
    gathers = [
        pltpu.async_copy(
            table_s.at[idx_v.at[pl.ds(i * ch, ch)]], rows_v.at[pl.ds(i * ch, ch)], sem
        )
        for i in range(_K)
    ]
    scatters = []
    for i in range(_K):
        gathers[i].wait()
        scatters.append(
            pltpu.async_copy(
                rows_v.at[pl.ds(i * ch, ch)], out_hbm.at[pl.ds(base + i * ch, ch)], ssem
            )
        )
    for s in scatters:
        s.wait()


def kernel(t, pe):
    return _pe_gather(t, pe)
